# baseline (speedup 1.0000x reference)
import functools

import jax
import jax.numpy as jnp
from jax import lax
from jax.experimental import pallas as pl
from jax.experimental.pallas import tpu as pltpu
from jax.experimental.pallas import tpu_sc as plsc

_NC = 2
_NS = 16
_NW = _NC * _NS

_B = 4096 * 50
_D = 64
_BPW = _B // _NW
_CHUNK = 1600
_NCHUNK = _BPW // _CHUNK


def _make_gather():
    mesh = plsc.VectorSubcoreMesh(
        core_axis_name="c",
        subcore_axis_name="s",
        num_cores=_NC,
        num_subcores=_NS,
    )

    @functools.partial(
        pl.kernel,
        out_type=jax.ShapeDtypeStruct((_B, _D), jnp.float32),
        mesh=mesh,
        scratch_types=[
            pltpu.SemaphoreType.DMA,
        ],
        compiler_params=pltpu.CompilerParams(use_tc_tiling_on_sc=False),
    )
    def gather(idx_hbm, table_hbm, out_hbm, sem):
        wid = lax.axis_index("s") * _NC + lax.axis_index("c")
        base = wid * _BPW

        for g in range(_NCHUNK):
            def inner(idx_c, rows_c, g=g):
                pltpu.sync_copy(idx_hbm.at[wid, g], idx_c)
                pltpu.async_copy(table_hbm.at[idx_c], rows_c, sem).wait()
                pltpu.sync_copy(
                    rows_c, out_hbm.at[pl.ds(base + g * _CHUNK, _CHUNK)])

            pl.run_scoped(
                inner,
                pltpu.VMEM((_CHUNK,), jnp.int32),
                pltpu.VMEM((_CHUNK, _D), jnp.float32),
            )

    return gather


_gather = _make_gather()


def kernel(x, weight):
    idx = x.reshape(_NW, _NCHUNK, _CHUNK)
    out = _gather(idx, weight)
    return out.reshape(x.shape[0], x.shape[1], _D)


# speedup vs baseline: 1.0007x; 1.0007x over previous
import functools

import jax
import jax.numpy as jnp
from jax import lax
from jax.experimental import pallas as pl
from jax.experimental.pallas import tpu as pltpu
from jax.experimental.pallas import tpu_sc as plsc

_NC = 2
_NS = 16
_NW = _NC * _NS

_B = 4096 * 50
_D = 64
_BPW = _B // _NW
_CHUNK = 800
_NCHUNK = _BPW // _CHUNK


def _make_gather():
    mesh = plsc.VectorSubcoreMesh(
        core_axis_name="c",
        subcore_axis_name="s",
        num_cores=_NC,
        num_subcores=_NS,
    )

    @functools.partial(
        pl.kernel,
        out_type=jax.ShapeDtypeStruct((_B, _D), jnp.float32),
        mesh=mesh,
        scratch_types=[
            pltpu.VMEM((_CHUNK,), jnp.int32),
            pltpu.VMEM((_CHUNK,), jnp.int32),
            pltpu.VMEM((_CHUNK, _D), jnp.float32),
            pltpu.VMEM((_CHUNK, _D), jnp.float32),
            pltpu.SemaphoreType.DMA,
            pltpu.SemaphoreType.DMA,
            pltpu.SemaphoreType.DMA,
            pltpu.SemaphoreType.DMA,
        ],
        compiler_params=pltpu.CompilerParams(use_tc_tiling_on_sc=False),
    )
    def gather(idx_hbm, table_hbm, out_hbm,
               idx0, idx1, rows0, rows1, gsem0, gsem1, osem0, osem1):
        wid = lax.axis_index("s") * _NC + lax.axis_index("c")
        base = wid * _BPW
        idx = (idx0, idx1)
        rows = (rows0, rows1)
        gsem = (gsem0, gsem1)
        osem = (osem0, osem1)

        gat = [None, None]
        outcp = [None, None]
        pltpu.sync_copy(idx_hbm.at[wid, 0], idx0)
        gat[0] = pltpu.async_copy(table_hbm.at[idx0], rows0, gsem0)
        for g in range(_NCHUNK):
            b = g & 1
            nb = 1 - b
            if g + 1 < _NCHUNK:
                pltpu.sync_copy(idx_hbm.at[wid, g + 1], idx[nb])
                if outcp[nb] is not None:
                    outcp[nb].wait()
                gat[nb] = pltpu.async_copy(
                    table_hbm.at[idx[nb]], rows[nb], gsem[nb])
            gat[b].wait()
            outcp[b] = pltpu.async_copy(
                rows[b], out_hbm.at[pl.ds(base + g * _CHUNK, _CHUNK)],
                osem[b])
        outcp[0].wait()
        outcp[1].wait()

    return gather


_gather = _make_gather()


def kernel(x, weight):
    idx = x.reshape(_NW, _NCHUNK, _CHUNK)
    out = _gather(idx, weight)
    return out.reshape(x.shape[0], x.shape[1], _D)
